# baseline (speedup 1.0000x reference)
import jax
import jax.numpy as jnp
from jax.experimental import pallas as pl

F32 = jnp.float32
N = 50000
E = 800000
BATCH = 4096
NCOL = 4
H = 16
EDIM = NCOL * H
NCLASS = 2
E_MP = E - BATCH
BE = 4096
E_PAD = 798720
BN = 1000
assert E_PAD % BE == 0 and N % BN == 0


def _full(shape):
    return pl.BlockSpec(shape, lambda *_: tuple(0 for _ in shape))


def _rows(bshape):
    return pl.BlockSpec(bshape, lambda i: (i,) + (0,) * (len(bshape) - 1))


def _node_prep_body(x_ref, wn_ref, bn_ref, wm0_ref, h0_ref, hm0_ref):
    h0 = jnp.maximum(x_ref[...] * wn_ref[...] + bn_ref[...], 0.0)
    h0_ref[...] = h0
    hm0_ref[...] = jnp.dot(h0, wm0_ref[...], preferred_element_type=F32)


def _attr_fma(ea, m):
    acc = ea[:, 0:1] * m[0:1, :]
    for c in range(1, NCOL):
        acc = acc + ea[:, c:c + 1] * m[c:c + 1, :]
    return acc


def _msg0_body(g_ref, ea_ref, m0_ref, c0_ref, msg_ref):
    msg_ref[...] = jnp.maximum(
        g_ref[...] + _attr_fma(ea_ref[...], m0_ref[...]) + c0_ref[...], 0.0)


def _hupd_body(h_ref, agg_ref, wu_ref, bu_ref, o_ref):
    o_ref[...] = jnp.maximum(
        h_ref[...]
        + jnp.dot(agg_ref[...], wu_ref[...], preferred_element_type=F32)
        + bu_ref[...], 0.0)


def _msg1_body(hs_ref, hd_ref, ea_ref, wnx_ref, m2_ref, cT_ref, wm1_ref,
               m1_ref, we1_ref, cM_ref, msg_ref):
    hs = hs_ref[...]
    ea = ea_ref[...]
    t = (jnp.dot(hs + hd_ref[...], wnx_ref[...], preferred_element_type=F32)
         + _attr_fma(ea, m2_ref[...]) + cT_ref[...])
    t = jnp.maximum(t, 0.0)
    m = (jnp.dot(hs, wm1_ref[...], preferred_element_type=F32)
         + _attr_fma(ea, m1_ref[...])
         + jnp.dot(t, we1_ref[...], preferred_element_type=F32)
         + cM_ref[...])
    msg_ref[...] = jnp.maximum(m, 0.0)


def _tail_body(h1s_ref, h1d_ref, h2s_ref, h2d_ref, ea_ref, wenc_ref, benc_ref,
               wnx0_ref, wee0_ref, be0_ref, wnx1_ref, wee1_ref, be1_ref,
               wc1a_ref, wc1b_ref, wc1c_ref, bc1_ref, wc2_ref, bc2_ref,
               out_ref):
    ea = ea_ref[...]
    wenc = wenc_ref[...]
    benc = benc_ref[...]
    e0 = jnp.concatenate(
        [ea[:, c:c + 1] * wenc[c:c + 1, :] + benc[c:c + 1, :]
         for c in range(NCOL)], axis=1)
    u1 = jnp.maximum(
        jnp.dot(h1s_ref[...] + h1d_ref[...], wnx0_ref[...],
                preferred_element_type=F32)
        + jnp.dot(e0, wee0_ref[...], preferred_element_type=F32)
        + be0_ref[...], 0.0)
    e1 = e0 + u1
    h2s = h2s_ref[...]
    h2d = h2d_ref[...]
    u2 = jnp.maximum(
        jnp.dot(h2s + h2d, wnx1_ref[...], preferred_element_type=F32)
        + jnp.dot(e1, wee1_ref[...], preferred_element_type=F32)
        + be1_ref[...], 0.0)
    e2 = e1 + u2
    z = (jnp.dot(h2s, wc1a_ref[...], preferred_element_type=F32)
         + jnp.dot(h2d, wc1b_ref[...], preferred_element_type=F32)
         + jnp.dot(e2, wc1c_ref[...], preferred_element_type=F32)
         + bc1_ref[...])
    z = jnp.maximum(z, 0.0)
    out_ref[...] = (jnp.dot(z, wc2_ref[...], preferred_element_type=F32)
                    + bc2_ref[...])


def kernel(x, edge_index, edge_attr, W_enc, b_enc, W_node, b_node,
           W_msg_0, W_edge_0, b_msg_0, W_upd_0, b_upd_0, W_enx_0, W_ee_0, b_e_0,
           W_msg_1, W_edge_1, b_msg_1, W_upd_1, b_upd_1, W_enx_1, W_ee_1, b_e_1,
           W_c1, b_c1, W_c2, b_c2):
    we0r = W_edge_0.reshape(NCOL, H, H)
    we1r = W_edge_1.reshape(NCOL, H, H)
    wee0r = W_ee_0.reshape(NCOL, H, EDIM)
    M0 = jnp.einsum('ch,chk->ck', W_enc, we0r)
    c0 = jnp.einsum('ch,chk->k', b_enc, we0r)
    M1 = jnp.einsum('ch,chk->ck', W_enc, we1r)
    c1 = jnp.einsum('ch,chk->k', b_enc, we1r)
    M2 = jnp.einsum('ch,chk->ck', W_enc, wee0r)
    c2 = jnp.einsum('ch,chk->k', b_enc, wee0r)
    cA = (c0 + b_msg_0).reshape(1, H)
    cT = (c2 + b_e_0).reshape(1, EDIM)
    cM = (c1 + b_msg_1).reshape(1, H)
    bn2 = b_node.reshape(1, H)
    bu0 = b_upd_0.reshape(1, H)
    bu1 = b_upd_1.reshape(1, H)
    be0 = b_e_0.reshape(1, EDIM)
    be1 = b_e_1.reshape(1, EDIM)
    bc1 = b_c1.reshape(1, H)
    bc2 = b_c2.reshape(1, NCLASS)
    Wc1a = W_c1[0:H, :]
    Wc1b = W_c1[H:2 * H, :]
    Wc1c = W_c1[2 * H:, :]

    tsrc = edge_index[0, :BATCH]
    tdst = edge_index[1, :BATCH]
    pad = E_PAD - E_MP
    src_mp = jnp.concatenate([edge_index[0, BATCH:],
                              jnp.zeros((pad,), jnp.int32)])
    dst_mp = jnp.concatenate([edge_index[1, BATCH:],
                              jnp.full((pad,), N, jnp.int32)])
    ea_mp = jnp.concatenate([edge_attr[BATCH:],
                             jnp.zeros((pad, NCOL), F32)])
    ea_tgt = edge_attr[:BATCH]

    h0, hm0 = pl.pallas_call(
        _node_prep_body,
        grid=(N // BN,),
        in_specs=[_rows((BN, 1)), _full((1, H)), _full((1, H)),
                  _full((H, H))],
        out_specs=[_rows((BN, H)), _rows((BN, H))],
        out_shape=[jax.ShapeDtypeStruct((N, H), F32),
                   jax.ShapeDtypeStruct((N, H), F32)],
    )(x, W_node, bn2, W_msg_0)

    g0 = jnp.take(hm0, src_mp, axis=0)
    msg0 = pl.pallas_call(
        _msg0_body,
        grid=(E_PAD // BE,),
        in_specs=[_rows((BE, H)), _rows((BE, NCOL)), _full((NCOL, H)),
                  _full((1, H))],
        out_specs=_rows((BE, H)),
        out_shape=jax.ShapeDtypeStruct((E_PAD, H), F32),
    )(g0, ea_mp, M0, cA)
    agg0 = jax.ops.segment_sum(msg0, dst_mp, num_segments=N)

    h1 = pl.pallas_call(
        _hupd_body,
        grid=(N // BN,),
        in_specs=[_rows((BN, H)), _rows((BN, H)), _full((H, H)),
                  _full((1, H))],
        out_specs=_rows((BN, H)),
        out_shape=jax.ShapeDtypeStruct((N, H), F32),
    )(h0, agg0, W_upd_0, bu0)

    hs1 = jnp.take(h1, src_mp, axis=0)
    hd1 = jnp.take(h1, dst_mp, axis=0, mode='clip')
    msg1 = pl.pallas_call(
        _msg1_body,
        grid=(E_PAD // BE,),
        in_specs=[_rows((BE, H)), _rows((BE, H)), _rows((BE, NCOL)),
                  _full((H, EDIM)), _full((NCOL, EDIM)), _full((1, EDIM)),
                  _full((H, H)), _full((NCOL, H)), _full((EDIM, H)),
                  _full((1, H))],
        out_specs=_rows((BE, H)),
        out_shape=jax.ShapeDtypeStruct((E_PAD, H), F32),
    )(hs1, hd1, ea_mp, W_enx_0, M2, cT, W_msg_1, M1, W_edge_1, cM)
    agg1 = jax.ops.segment_sum(msg1, dst_mp, num_segments=N)

    h2 = pl.pallas_call(
        _hupd_body,
        grid=(N // BN,),
        in_specs=[_rows((BN, H)), _rows((BN, H)), _full((H, H)),
                  _full((1, H))],
        out_specs=_rows((BN, H)),
        out_shape=jax.ShapeDtypeStruct((N, H), F32),
    )(h1, agg1, W_upd_1, bu1)

    h1s = jnp.take(h1, tsrc, axis=0)
    h1d = jnp.take(h1, tdst, axis=0)
    h2s = jnp.take(h2, tsrc, axis=0)
    h2d = jnp.take(h2, tdst, axis=0)
    out = pl.pallas_call(
        _tail_body,
        grid=(1,),
        in_specs=[_full((BATCH, H)), _full((BATCH, H)), _full((BATCH, H)),
                  _full((BATCH, H)), _full((BATCH, NCOL)), _full((NCOL, H)),
                  _full((NCOL, H)), _full((H, EDIM)), _full((EDIM, EDIM)),
                  _full((1, EDIM)), _full((H, EDIM)), _full((EDIM, EDIM)),
                  _full((1, EDIM)), _full((H, H)), _full((H, H)),
                  _full((EDIM, H)), _full((1, H)), _full((H, NCLASS)),
                  _full((1, NCLASS))],
        out_specs=_full((BATCH, NCLASS)),
        out_shape=jax.ShapeDtypeStruct((BATCH, NCLASS), F32),
    )(h1s, h1d, h2s, h2d, ea_tgt, W_enc, b_enc, W_enx_0, W_ee_0, be0,
      W_enx_1, W_ee_1, be1, Wc1a, Wc1b, Wc1c, bc1, W_c2, bc2)
    return out


# speedup vs baseline: 6.6243x; 6.6243x over previous
import functools

import jax
import jax.numpy as jnp
from jax import lax
from jax.experimental import pallas as pl
from jax.experimental.pallas import tpu as pltpu
from jax.experimental.pallas import tpu_sc as plsc

F32 = jnp.float32
N = 50000
E = 800000
BATCH = 4096
NCOL = 4
H = 16
EDIM = NCOL * H
NCLASS = 2
E_MP = E - BATCH
BE = 4096
BN = 1000
CH = 128
NCH = E_MP // CH
NROW_T = N // 16
assert NCH * CH == E_MP and NROW_T * 16 == N

_vmesh = plsc.VectorSubcoreMesh(core_axis_name="c", subcore_axis_name="s")
_sc_params = pltpu.CompilerParams(use_tc_tiling_on_sc=False)


def _sc_gather1(table, idx_flat):
    @functools.partial(
        pl.kernel, mesh=_vmesh, compiler_params=_sc_params,
        out_type=jax.ShapeDtypeStruct((E_MP, H), F32))
    def k(tab_hbm, idx_hbm, out_hbm):
        def body(i_vmem, o_vmem):
            pltpu.sync_copy(tab_hbm.at[i_vmem.at[0]], o_vmem)
        pltpu.emit_pipeline(
            body, grid=(NCH,),
            in_specs=[pl.BlockSpec((1, CH), lambda i: (0, i))],
            out_specs=[pl.BlockSpec((CH, H), lambda i: (i, 0))],
            core_axis_name=("c", "s"),
            dimension_semantics=(pltpu.PARALLEL,),
        )(idx_hbm, out_hbm)
    return k(table, idx_flat)


def _sc_gather2(table, si_flat, di_flat):
    @functools.partial(
        pl.kernel, mesh=_vmesh, compiler_params=_sc_params,
        out_type=[jax.ShapeDtypeStruct((E_MP, H), F32),
                  jax.ShapeDtypeStruct((E_MP, H), F32)])
    def k(tab_hbm, si_hbm, di_hbm, hs_hbm, hd_hbm):
        def body(si_vmem, di_vmem, hs_vmem, hd_vmem):
            pltpu.sync_copy(tab_hbm.at[si_vmem.at[0]], hs_vmem)
            pltpu.sync_copy(tab_hbm.at[di_vmem.at[0]], hd_vmem)
        pltpu.emit_pipeline(
            body, grid=(NCH,),
            in_specs=[pl.BlockSpec((1, CH), lambda i: (0, i)),
                      pl.BlockSpec((1, CH), lambda i: (0, i))],
            out_specs=[pl.BlockSpec((CH, H), lambda i: (i, 0)),
                       pl.BlockSpec((CH, H), lambda i: (i, 0))],
            core_axis_name=("c", "s"),
            dimension_semantics=(pltpu.PARALLEL,),
        )(si_hbm, di_hbm, hs_hbm, hd_hbm)
    return k(table, si_flat, di_flat)


def _sc_scatter_add(msg, dst_flat):
    @functools.partial(
        pl.kernel, mesh=_vmesh, compiler_params=_sc_params,
        out_type=jax.ShapeDtypeStruct((2 * N, H), F32),
        scratch_types=[pltpu.VMEM((NROW_T, H), F32),
                       pltpu.VMEM_SHARED((N, H), F32)])
    def k(msg_hbm, dst_hbm, p_hbm, zb, shared):
        c = lax.axis_index("c")
        s = lax.axis_index("s")

        @pl.loop(0, NROW_T)
        def _(r):
            zb[r, :] = jnp.zeros((H,), F32)

        pltpu.sync_copy(zb, shared.at[pl.ds(s * NROW_T, NROW_T)])
        plsc.subcore_barrier()

        def body(m_vmem, i_vmem):
            pltpu.sync_copy(m_vmem, shared.at[i_vmem.at[0]], add=True)

        pltpu.emit_pipeline(
            body, grid=(NCH,),
            in_specs=[pl.BlockSpec((CH, H), lambda i: (i, 0)),
                      pl.BlockSpec((1, CH), lambda i: (0, i))],
            core_axis_name=("c", "s"),
            dimension_semantics=(pltpu.PARALLEL,),
        )(msg_hbm, dst_hbm)
        plsc.subcore_barrier()
        pltpu.sync_copy(shared.at[pl.ds(s * NROW_T, NROW_T)],
                        p_hbm.at[pl.ds(c * N + s * NROW_T, NROW_T)])
    return k(msg, dst_flat)


def _full(shape):
    return pl.BlockSpec(shape, lambda *_: tuple(0 for _ in shape))


def _rows(bshape, off=0):
    return pl.BlockSpec(bshape, lambda i: (i + off,) + (0,) * (len(bshape) - 1))


def _node_prep_body(x_ref, wn_ref, bn_ref, wm0_ref, h0_ref, hm0_ref):
    h0 = jnp.maximum(x_ref[...] * wn_ref[...] + bn_ref[...], 0.0)
    h0_ref[...] = h0
    hm0_ref[...] = jnp.dot(h0, wm0_ref[...], preferred_element_type=F32)


def _attr_fma(ea, m):
    acc = ea[:, 0:1] * m[0:1, :]
    for c in range(1, NCOL):
        acc = acc + ea[:, c:c + 1] * m[c:c + 1, :]
    return acc


def _msg0_body(g_ref, ea_ref, m0_ref, c0_ref, msg_ref):
    msg_ref[...] = jnp.maximum(
        g_ref[...] + _attr_fma(ea_ref[...], m0_ref[...]) + c0_ref[...], 0.0)


def _hupd_body(h_ref, pa_ref, pb_ref, wu_ref, bu_ref, o_ref):
    o_ref[...] = jnp.maximum(
        h_ref[...]
        + jnp.dot(pa_ref[...] + pb_ref[...], wu_ref[...],
                  preferred_element_type=F32)
        + bu_ref[...], 0.0)


def _msg1_body(hs_ref, hd_ref, ea_ref, wnx_ref, m2_ref, cT_ref, wm1_ref,
               m1_ref, we1_ref, cM_ref, msg_ref):
    hs = hs_ref[...]
    ea = ea_ref[...]
    t = (jnp.dot(hs + hd_ref[...], wnx_ref[...], preferred_element_type=F32)
         + _attr_fma(ea, m2_ref[...]) + cT_ref[...])
    t = jnp.maximum(t, 0.0)
    m = (jnp.dot(hs, wm1_ref[...], preferred_element_type=F32)
         + _attr_fma(ea, m1_ref[...])
         + jnp.dot(t, we1_ref[...], preferred_element_type=F32)
         + cM_ref[...])
    msg_ref[...] = jnp.maximum(m, 0.0)


def _tail_body(h1s_ref, h1d_ref, h2s_ref, h2d_ref, ea_ref, wenc_ref, benc_ref,
               wnx0_ref, wee0_ref, be0_ref, wnx1_ref, wee1_ref, be1_ref,
               wc1a_ref, wc1b_ref, wc1c_ref, bc1_ref, wc2_ref, bc2_ref,
               out_ref):
    ea = ea_ref[...]
    wenc = wenc_ref[...]
    benc = benc_ref[...]
    e0 = jnp.concatenate(
        [ea[:, c:c + 1] * wenc[c:c + 1, :] + benc[c:c + 1, :]
         for c in range(NCOL)], axis=1)
    u1 = jnp.maximum(
        jnp.dot(h1s_ref[...] + h1d_ref[...], wnx0_ref[...],
                preferred_element_type=F32)
        + jnp.dot(e0, wee0_ref[...], preferred_element_type=F32)
        + be0_ref[...], 0.0)
    e1 = e0 + u1
    h2s = h2s_ref[...]
    h2d = h2d_ref[...]
    u2 = jnp.maximum(
        jnp.dot(h2s + h2d, wnx1_ref[...], preferred_element_type=F32)
        + jnp.dot(e1, wee1_ref[...], preferred_element_type=F32)
        + be1_ref[...], 0.0)
    e2 = e1 + u2
    z = (jnp.dot(h2s, wc1a_ref[...], preferred_element_type=F32)
         + jnp.dot(h2d, wc1b_ref[...], preferred_element_type=F32)
         + jnp.dot(e2, wc1c_ref[...], preferred_element_type=F32)
         + bc1_ref[...])
    z = jnp.maximum(z, 0.0)
    out_ref[...] = (jnp.dot(z, wc2_ref[...], preferred_element_type=F32)
                    + bc2_ref[...])


def kernel(x, edge_index, edge_attr, W_enc, b_enc, W_node, b_node,
           W_msg_0, W_edge_0, b_msg_0, W_upd_0, b_upd_0, W_enx_0, W_ee_0, b_e_0,
           W_msg_1, W_edge_1, b_msg_1, W_upd_1, b_upd_1, W_enx_1, W_ee_1, b_e_1,
           W_c1, b_c1, W_c2, b_c2):
    we0r = W_edge_0.reshape(NCOL, H, H)
    we1r = W_edge_1.reshape(NCOL, H, H)
    wee0r = W_ee_0.reshape(NCOL, H, EDIM)
    M0 = jnp.einsum('ch,chk->ck', W_enc, we0r)
    c0 = jnp.einsum('ch,chk->k', b_enc, we0r)
    M1 = jnp.einsum('ch,chk->ck', W_enc, we1r)
    c1 = jnp.einsum('ch,chk->k', b_enc, we1r)
    M2 = jnp.einsum('ch,chk->ck', W_enc, wee0r)
    c2 = jnp.einsum('ch,chk->k', b_enc, wee0r)
    cA = (c0 + b_msg_0).reshape(1, H)
    cT = (c2 + b_e_0).reshape(1, EDIM)
    cM = (c1 + b_msg_1).reshape(1, H)
    bn2 = b_node.reshape(1, H)
    bu0 = b_upd_0.reshape(1, H)
    bu1 = b_upd_1.reshape(1, H)
    be0 = b_e_0.reshape(1, EDIM)
    be1 = b_e_1.reshape(1, EDIM)
    bc1 = b_c1.reshape(1, H)
    bc2 = b_c2.reshape(1, NCLASS)
    Wc1a = W_c1[0:H, :]
    Wc1b = W_c1[H:2 * H, :]
    Wc1c = W_c1[2 * H:, :]

    tsrc = edge_index[0, :BATCH]
    tdst = edge_index[1, :BATCH]
    src_flat = edge_index[0, BATCH:].reshape(1, E_MP)
    dst_flat = edge_index[1, BATCH:].reshape(1, E_MP)
    ea_tgt = edge_attr[:BATCH]

    h0, hm0 = pl.pallas_call(
        _node_prep_body,
        grid=(N // BN,),
        in_specs=[_rows((BN, 1)), _full((1, H)), _full((1, H)),
                  _full((H, H))],
        out_specs=[_rows((BN, H)), _rows((BN, H))],
        out_shape=[jax.ShapeDtypeStruct((N, H), F32),
                   jax.ShapeDtypeStruct((N, H), F32)],
    )(x, W_node, bn2, W_msg_0)

    g0 = _sc_gather1(hm0, src_flat)
    nblk = -(-E_MP // BE)
    msg0 = pl.pallas_call(
        _msg0_body,
        grid=(nblk,),
        in_specs=[_rows((BE, H)), _rows((BE, NCOL), off=1), _full((NCOL, H)),
                  _full((1, H))],
        out_specs=_rows((BE, H)),
        out_shape=jax.ShapeDtypeStruct((E_MP, H), F32),
    )(g0, edge_attr, M0, cA)
    p0 = _sc_scatter_add(msg0, dst_flat)

    h1 = pl.pallas_call(
        _hupd_body,
        grid=(N // BN,),
        in_specs=[_rows((BN, H)), _rows((BN, H)), _rows((BN, H), off=N // BN),
                  _full((H, H)), _full((1, H))],
        out_specs=_rows((BN, H)),
        out_shape=jax.ShapeDtypeStruct((N, H), F32),
    )(h0, p0, p0, W_upd_0, bu0)

    hs1, hd1 = _sc_gather2(h1, src_flat, dst_flat)
    msg1 = pl.pallas_call(
        _msg1_body,
        grid=(nblk,),
        in_specs=[_rows((BE, H)), _rows((BE, H)), _rows((BE, NCOL), off=1),
                  _full((H, EDIM)), _full((NCOL, EDIM)), _full((1, EDIM)),
                  _full((H, H)), _full((NCOL, H)), _full((EDIM, H)),
                  _full((1, H))],
        out_specs=_rows((BE, H)),
        out_shape=jax.ShapeDtypeStruct((E_MP, H), F32),
    )(hs1, hd1, edge_attr, W_enx_0, M2, cT, W_msg_1, M1, W_edge_1, cM)
    p1 = _sc_scatter_add(msg1, dst_flat)

    h2 = pl.pallas_call(
        _hupd_body,
        grid=(N // BN,),
        in_specs=[_rows((BN, H)), _rows((BN, H)), _rows((BN, H), off=N // BN),
                  _full((H, H)), _full((1, H))],
        out_specs=_rows((BN, H)),
        out_shape=jax.ShapeDtypeStruct((N, H), F32),
    )(h1, p1, p1, W_upd_1, bu1)

    h1s = jnp.take(h1, tsrc, axis=0)
    h1d = jnp.take(h1, tdst, axis=0)
    h2s = jnp.take(h2, tsrc, axis=0)
    h2d = jnp.take(h2, tdst, axis=0)
    out = pl.pallas_call(
        _tail_body,
        grid=(1,),
        in_specs=[_full((BATCH, H)), _full((BATCH, H)), _full((BATCH, H)),
                  _full((BATCH, H)), _full((BATCH, NCOL)), _full((NCOL, H)),
                  _full((NCOL, H)), _full((H, EDIM)), _full((EDIM, EDIM)),
                  _full((1, EDIM)), _full((H, EDIM)), _full((EDIM, EDIM)),
                  _full((1, EDIM)), _full((H, H)), _full((H, H)),
                  _full((EDIM, H)), _full((1, H)), _full((H, NCLASS)),
                  _full((1, NCLASS))],
        out_specs=_full((BATCH, NCLASS)),
        out_shape=jax.ShapeDtypeStruct((BATCH, NCLASS), F32),
    )(h1s, h1d, h2s, h2d, ea_tgt, W_enc, b_enc, W_enx_0, W_ee_0, be0,
      W_enx_1, W_ee_1, be1, Wc1a, Wc1b, Wc1c, bc1, W_c2, bc2)
    return out
